# Initial kernel scaffold; baseline (speedup 1.0000x reference)
import jax
import jax.numpy as jnp
from jax.experimental import pallas as pl


def kernel(x, emb_weight):
    raise NotImplementedError("write your pallas kernel here")


# speedup vs baseline: 4.0405x; 4.0405x over previous
import functools
import math

import jax
import jax.numpy as jnp
from jax import lax
from jax.experimental import pallas as pl
from jax.experimental.pallas import tpu as pltpu
from jax.experimental.pallas import tpu_sc as plsc

D_MODEL = 32
SCALE = math.sqrt(D_MODEL)

_info = plsc.get_sparse_core_info()
NC, NS, L = _info.num_cores, _info.num_subcores, _info.num_lanes
NW = NC * NS

GW = 128
G = 8
C = G * GW


def _emb_kernel(nchunks, x2d, table, out, idx_v, rows_v, sem):
  wid = lax.axis_index("s") * NC + lax.axis_index("c")
  worker_row0 = wid * (nchunks * C)

  def chunk_body(c, _):
    base = worker_row0 + c * C
    pltpu.sync_copy(x2d.at[pl.ds(pl.multiple_of(base // GW, 8), G)], idx_v)
    copies = [
        pltpu.async_copy(table.at[idx_v.at[j]],
                         rows_v.at[pl.ds(j * GW, GW)], sem)
        for j in range(G)
    ]
    for d in copies:
      d.wait()

    def scale_body(i, _):
      v0 = rows_v[i, pl.ds(0, L)]
      v1 = rows_v[i, pl.ds(L, L)]
      rows_v[i, pl.ds(0, L)] = v0 * SCALE
      rows_v[i, pl.ds(L, L)] = v1 * SCALE
      return 0

    lax.fori_loop(0, C, scale_body, 0)

    pltpu.sync_copy(rows_v, out.at[pl.ds(base, C)])
    return 0

  lax.fori_loop(0, nchunks, chunk_body, 0)


def kernel(x, emb_weight):
  B = x.shape[0] * x.shape[1]
  assert B % (NW * C) == 0
  nchunks = B // (NW * C)
  x2d = x.reshape(B // GW, GW).astype(jnp.int32)

  mesh = plsc.VectorSubcoreMesh(core_axis_name="c", subcore_axis_name="s")
  run = pl.kernel(
      functools.partial(_emb_kernel, nchunks),
      out_type=jax.ShapeDtypeStruct((B, D_MODEL), jnp.float32),
      mesh=mesh,
      scratch_types=[
          pltpu.VMEM((G, GW), jnp.int32),
          pltpu.VMEM((C, D_MODEL), jnp.float32),
          pltpu.SemaphoreType.DMA,
      ],
      compiler_params=pltpu.CompilerParams(use_tc_tiling_on_sc=False),
  )
  out = run(x2d, emb_weight)
  return out.reshape(x.shape[0], x.shape[1], D_MODEL)


# speedup vs baseline: 4.9115x; 1.2156x over previous
import functools
import math

import jax
import jax.numpy as jnp
from jax import lax
from jax.experimental import pallas as pl
from jax.experimental.pallas import tpu as pltpu
from jax.experimental.pallas import tpu_sc as plsc

D_MODEL = 32
SCALE = math.sqrt(D_MODEL)

_info = plsc.get_sparse_core_info()
NC, NS, L = _info.num_cores, _info.num_subcores, _info.num_lanes
NW = NC * NS

GW = 128
G = 8
C = G * GW


def _emb_kernel(nchunks, x2d, table, out, idx_v, rows_v, gsem0, gsem1):
  gsem = (gsem0, gsem1)
  wid = lax.axis_index("s") * NC + lax.axis_index("c")
  worker_row0 = wid * (nchunks * C)

  def row_base(c):
    return worker_row0 + c * C

  def fire(c, b):
    base = row_base(c)
    pltpu.sync_copy(x2d.at[pl.ds(pl.multiple_of(base // GW, 8), G)],
                    idx_v.at[b])
    for j in range(G):
      pltpu.async_copy(table.at[idx_v.at[b].at[j]],
                       rows_v.at[b].at[pl.ds(j * GW, GW)], gsem[b])

  def wait_gathers(c, b):
    pltpu.make_async_copy(out.at[pl.ds(row_base(c), C)], rows_v.at[b],
                          gsem[b]).wait()

  def scale(b):
    rb = rows_v.at[b]

    @plsc.parallel_loop(0, C, step=1, unroll=8)
    def _(i):
      rb[i, pl.ds(0, L)] = rb[i, pl.ds(0, L)] * SCALE
      rb[i, pl.ds(L, L)] = rb[i, pl.ds(L, L)] * SCALE

  fire(0, 0)

  def outer(t, _):
    for b in range(2):
      c = 2 * t + b

      @pl.when(c + 1 < nchunks)
      def _():
        fire(c + 1, 1 - b)

      wait_gathers(c, b)
      scale(b)
      pltpu.sync_copy(rows_v.at[b], out.at[pl.ds(row_base(c), C)])
    return 0

  lax.fori_loop(0, nchunks // 2, outer, 0)


def kernel(x, emb_weight):
  B = x.shape[0] * x.shape[1]
  assert B % (NW * C * 2) == 0
  nchunks = B // (NW * C)
  x2d = x.reshape(B // GW, GW).astype(jnp.int32)

  mesh = plsc.VectorSubcoreMesh(core_axis_name="c", subcore_axis_name="s")
  run = pl.kernel(
      functools.partial(_emb_kernel, nchunks),
      out_type=jax.ShapeDtypeStruct((B, D_MODEL), jnp.float32),
      mesh=mesh,
      scratch_types=[
          pltpu.VMEM((2, G, GW), jnp.int32),
          pltpu.VMEM((2, C, D_MODEL), jnp.float32),
          pltpu.SemaphoreType.DMA,
          pltpu.SemaphoreType.DMA,
      ],
      compiler_params=pltpu.CompilerParams(use_tc_tiling_on_sc=False),
  )
  out = run(x2d, emb_weight)
  return out.reshape(x.shape[0], x.shape[1], D_MODEL)


# speedup vs baseline: 6.4096x; 1.3050x over previous
import functools
import math

import jax
import jax.numpy as jnp
from jax import lax
from jax.experimental import pallas as pl
from jax.experimental.pallas import tpu as pltpu
from jax.experimental.pallas import tpu_sc as plsc

D_MODEL = 32
SCALE = math.sqrt(D_MODEL)

_info = plsc.get_sparse_core_info()
NC, NS, L = _info.num_cores, _info.num_subcores, _info.num_lanes
NW = NC * NS

GW = 128
JB = 8
JH = 4


def _emb_kernel(nblk, xT2, table, out5, idx_v, rows_v, tbuf,
                gsem0, gsem1, wsem0, wsem1):
  gsem = (gsem0, gsem1)
  wsem = (wsem0, wsem1)
  wid = lax.axis_index("s") * NC + lax.axis_index("c")
  q0 = wid * (nblk * JB)

  def stage_idx(blk):
    bi = lax.rem(blk, 2)
    base = q0 + blk * JB
    pltpu.sync_copy(xT2.at[pl.ds(pl.multiple_of(base, 8), JB)],
                    idx_v.at[bi])

  def fire_gathers(blk, h, gb):
    bi = lax.rem(blk, 2)
    for jj in range(JH):
      pltpu.async_copy(table.at[idx_v.at[bi].at[h * JH + jj]],
                       rows_v.at[gb].at[pl.ds(jj * GW, GW)], gsem[gb])

  def wait_gathers(gb):
    pltpu.make_async_copy(table.at[pl.ds(0, JH * GW)], rows_v.at[gb],
                          gsem[gb]).wait()

  def t_bq(blk, h, jj):
    q = q0 + blk * JB + h * JH + jj
    return q >> 7, q & 127

  def transpose_scale(gb, jj):
    rv = rows_v.at[gb]
    tb = tbuf.at[gb]

    @plsc.parallel_loop(0, D_MODEL, step=1, unroll=2)
    def _(d):
      iota = lax.iota(jnp.int32, L)
      col = jnp.full((L,), d, dtype=jnp.int32)
      for bg in range(GW // L):
        v = plsc.load_gather(rv, [iota + (jj * GW + bg * L), col])
        tb[jj * D_MODEL + d, pl.ds(bg * L, L)] = v * SCALE

  def write_copies(blk, h, gb, jj):
    t, bq = t_bq(blk, h, jj)
    return [
        pltpu.make_async_copy(
            tbuf.at[gb].at[pl.ds(jj * D_MODEL + dq * 8, 8)],
            out5.at[t, dq, bq], wsem[gb])
        for dq in range(D_MODEL // 8)
    ]

  def fire_writes(blk, h, gb):
    for jj in range(JH):
      for c in write_copies(blk, h, gb, jj):
        c.start()

  def drain_writes(blk, h, gb):
    for jj in range(JH):
      for c in write_copies(blk, h, gb, jj):
        c.wait()

  stage_idx(0)
  fire_gathers(0, 0, 0)

  def outer(blk, _):
    for h in range(2):
      gb = h
      if h == 0:
        @pl.when(blk + 1 < nblk)
        def _():
          stage_idx(blk + 1)
        fire_gathers(blk, 1, 1)
      else:
        @pl.when(blk + 1 < nblk)
        def _():
          fire_gathers(blk + 1, 0, 0)

      wait_gathers(gb)

      @pl.when(blk >= 1)
      def _():
        drain_writes(blk - 1, h, gb)

      for jj in range(JH):
        transpose_scale(gb, jj)
      fire_writes(blk, h, gb)
    return 0

  lax.fori_loop(0, nblk, outer, 0)
  drain_writes(nblk - 1, 0, 0)
  drain_writes(nblk - 1, 1, 1)


def kernel(x, emb_weight):
  B = x.shape[0] * x.shape[1]
  T = x.shape[1]
  NB = x.shape[0]
  Q = B // GW
  assert Q % (NW * JB) == 0 and NB % GW == 0 and T % 8 == 0
  nblk = Q // (NW * JB)
  xT2 = x.T.reshape(Q, GW).astype(jnp.int32)

  mesh = plsc.VectorSubcoreMesh(core_axis_name="c", subcore_axis_name="s")
  run = pl.kernel(
      functools.partial(_emb_kernel, nblk),
      out_type=jax.ShapeDtypeStruct(
          (T, D_MODEL // 8, NB // GW, 8, GW), jnp.float32),
      mesh=mesh,
      scratch_types=[
          pltpu.VMEM((2, JB, GW), jnp.int32),
          pltpu.VMEM((2, JH * GW, D_MODEL), jnp.float32),
          pltpu.VMEM((2, JH * D_MODEL, GW), jnp.float32),
          pltpu.SemaphoreType.DMA,
          pltpu.SemaphoreType.DMA,
          pltpu.SemaphoreType.DMA,
          pltpu.SemaphoreType.DMA,
      ],
      compiler_params=pltpu.CompilerParams(use_tc_tiling_on_sc=False,
                                           needs_layout_passes=False),
  )
  out5 = run(xT2, emb_weight)
  return out5.transpose(2, 4, 0, 1, 3).reshape(NB, T, D_MODEL)
